# baseline (speedup 1.0000x reference)
import functools

import jax
import jax.numpy as jnp
from jax import lax
from jax.experimental import pallas as pl
from jax.experimental.pallas import tpu as pltpu
from jax.experimental.pallas import tpu_sc as plsc

N_ROWS = 32768
D = 2048
NC, NS = 2, 16
NW = NC * NS
B_PER_W = N_ROWS // NW
CHUNK = 16
N_CHUNKS = B_PER_W // CHUNK
NBUF = 3


PICK_SEG = 128
N_PICK_SEGS = B_PER_W // PICK_SEG


def _sc_gather(table, table_flat, idx_flat, tgt_flat):
    mesh = plsc.VectorSubcoreMesh(core_axis_name="c", subcore_axis_name="s")

    @functools.partial(
        pl.kernel,
        mesh=mesh,
        out_type=(
            jax.ShapeDtypeStruct((N_ROWS, D), jnp.float32),
            jax.ShapeDtypeStruct((N_ROWS,), jnp.float32),
        ),
        scratch_types=[
            pltpu.VMEM((B_PER_W,), jnp.int32),
            pltpu.VMEM((B_PER_W,), jnp.int32),
            pltpu.VMEM((B_PER_W,), jnp.int32),
            pltpu.VMEM((B_PER_W,), jnp.float32),
            pltpu.SemaphoreType.DMA,
        ]
        + [pltpu.VMEM((CHUNK, D), jnp.float32) for _ in range(NBUF)]
        + [pltpu.SemaphoreType.DMA for _ in range(2 * NBUF)],
    )
    def gather_kernel(
        table_hbm, tflat_hbm, idx_hbm, tgt_hbm, out_hbm, picked_hbm,
        idx_v, tgt_v, pick_idx_v, picked_v, sem_p, *scratch
    ):
        bufs = scratch[:NBUF]
        sem_g = scratch[NBUF : 2 * NBUF]
        sem_s = scratch[2 * NBUF :]
        wid = lax.axis_index("s") * NC + lax.axis_index("c")
        base = wid * B_PER_W
        pltpu.sync_copy(idx_hbm.at[pl.ds(base, B_PER_W)], idx_v)
        pltpu.sync_copy(tgt_hbm.at[pl.ds(base, B_PER_W)], tgt_v)

        def pick_idx_body(i, carry):
            sl = pl.ds(i * 16, 16)
            pick_idx_v[sl] = idx_v[sl] * D + tgt_v[sl]
            return carry

        lax.fori_loop(0, B_PER_W // 16, pick_idx_body, 0)
        for k in range(N_PICK_SEGS):
            sl = pl.ds(k * PICK_SEG, PICK_SEG)
            pltpu.async_copy(
                tflat_hbm.at[pick_idx_v.at[sl]], picked_v.at[sl], sem_p
            )

        def gather_chunk(i, b):
            pltpu.async_copy(
                table_hbm.at[idx_v.at[pl.ds(i * CHUNK, CHUNK)]], bufs[b], sem_g[b]
            )

        def wait_gather(i, b):
            pltpu.make_async_copy(
                table_hbm.at[idx_v.at[pl.ds(i * CHUNK, CHUNK)]], bufs[b], sem_g[b]
            ).wait()

        def store_chunk(i, b):
            pltpu.async_copy(
                bufs[b], out_hbm.at[pl.ds(base + i * CHUNK, CHUNK)], sem_s[b]
            )

        def wait_store(i, b):
            pltpu.make_async_copy(
                bufs[b], out_hbm.at[pl.ds(base + i * CHUNK, CHUNK)], sem_s[b]
            ).wait()

        gather_chunk(0, 0)

        N_MAIN = (N_CHUNKS - 1) // NBUF * NBUF

        def body(g, carry):
            for b in range(NBUF):
                j = g * NBUF + b
                bn = (b + 1) % NBUF
                wait_gather(j, b)
                store_chunk(j, b)

                @pl.when(j - (NBUF - 1) >= 0)
                def _():
                    wait_store(j - (NBUF - 1), bn)

                gather_chunk(j + 1, bn)

            return carry

        lax.fori_loop(0, N_MAIN // NBUF, body, 0)

        for j in range(N_MAIN, N_CHUNKS):
            b = j % NBUF
            wait_gather(j, b)
            store_chunk(j, b)
        for k in range(N_PICK_SEGS):
            sl = pl.ds(k * PICK_SEG, PICK_SEG)
            pltpu.make_async_copy(
                tflat_hbm.at[pick_idx_v.at[sl]], picked_v.at[sl], sem_p
            ).wait()
        pltpu.sync_copy(picked_v, picked_hbm.at[pl.ds(base, B_PER_W)])
        for j in range(N_CHUNKS - NBUF, N_CHUNKS):
            wait_store(j, j % NBUF)

    return gather_kernel(table, table_flat, idx_flat, tgt_flat)


ROWS_BLK = 256
N_BLKS = N_ROWS // ROWS_BLK


def _tc_loss_kernel(picked_ref, logits_ref, acc_ref):
    i = pl.program_id(0)
    blk = logits_ref[...]
    m = jnp.max(blk, axis=1, keepdims=True)
    lse = jnp.log(jnp.sum(jnp.exp(blk - m), axis=1, keepdims=True)) + m
    part = jnp.sum(lse) - jnp.sum(picked_ref[0, 0, :])

    @pl.when(i == 0)
    def _():
        acc_ref[0, 0] = 0.0

    acc_ref[0, 0] += part

    @pl.when(i == N_BLKS - 1)
    def _():
        acc_ref[0, 0] = acc_ref[0, 0] / N_ROWS


def _tc_loss(logits, picked):
    picked3 = picked.reshape(N_BLKS, 1, ROWS_BLK)
    acc = pl.pallas_call(
        _tc_loss_kernel,
        grid=(N_BLKS,),
        in_specs=[
            pl.BlockSpec((1, 1, ROWS_BLK), lambda i: (i, 0, 0)),
            pl.BlockSpec((ROWS_BLK, D), lambda i: (i, 0)),
        ],
        out_specs=pl.BlockSpec(
            (1, 1), lambda i: (0, 0), memory_space=pltpu.SMEM
        ),
        out_shape=jax.ShapeDtypeStruct((1, 1), jnp.float32),
    )(picked3, logits)
    return acc[0, 0]


def kernel(x, targets, token_embedding_table):
    idx_flat = x.reshape(N_ROWS)
    tgt_flat = targets.reshape(N_ROWS)
    table_flat = token_embedding_table.reshape(-1)
    logits, picked = _sc_gather(
        token_embedding_table, table_flat, idx_flat, tgt_flat
    )
    loss = _tc_loss(logits, picked)
    return (logits, loss)


# speedup vs baseline: 3.4016x; 3.4016x over previous
import functools

import jax
import jax.numpy as jnp
from jax import lax
from jax.experimental import pallas as pl
from jax.experimental.pallas import tpu as pltpu
from jax.experimental.pallas import tpu_sc as plsc

N_ROWS = 32768
D = 2048
NC, NS = 2, 16
NW = NC * NS
B_PER_W = N_ROWS // NW
CHUNK = 16
N_CHUNKS = B_PER_W // CHUNK
NBUF = 3


def _sc_gather(table, idx_flat, tgt_flat):
    mesh = plsc.VectorSubcoreMesh(core_axis_name="c", subcore_axis_name="s")

    @functools.partial(
        pl.kernel,
        mesh=mesh,
        out_type=(
            jax.ShapeDtypeStruct((N_ROWS, D), jnp.float32),
            jax.ShapeDtypeStruct((N_ROWS,), jnp.float32),
        ),
        scratch_types=[
            pltpu.VMEM((B_PER_W,), jnp.int32),
            pltpu.VMEM((B_PER_W,), jnp.int32),
            pltpu.VMEM((B_PER_W,), jnp.float32),
        ]
        + [pltpu.VMEM((CHUNK, D), jnp.float32) for _ in range(NBUF)]
        + [pltpu.SemaphoreType.DMA for _ in range(2 * NBUF)],
    )
    def gather_kernel(
        table_hbm, idx_hbm, tgt_hbm, out_hbm, picked_hbm,
        idx_v, tgt_v, picked_v, *scratch
    ):
        bufs = scratch[:NBUF]
        sem_g = scratch[NBUF : 2 * NBUF]
        sem_s = scratch[2 * NBUF :]
        wid = lax.axis_index("s") * NC + lax.axis_index("c")
        base = wid * B_PER_W
        pltpu.sync_copy(idx_hbm.at[pl.ds(base, B_PER_W)], idx_v)
        pltpu.sync_copy(tgt_hbm.at[pl.ds(base, B_PER_W)], tgt_v)

        def gather_chunk(i, b):
            pltpu.async_copy(
                table_hbm.at[idx_v.at[pl.ds(i * CHUNK, CHUNK)]],
                bufs[b],
                sem_g[b],
            )

        def wait_gather(i, b):
            pltpu.make_async_copy(
                table_hbm.at[idx_v.at[pl.ds(i * CHUNK, CHUNK)]],
                bufs[b],
                sem_g[b],
            ).wait()

        def store_chunk(i, b):
            pltpu.async_copy(
                bufs[b],
                out_hbm.at[pl.ds(base + i * CHUNK, CHUNK)],
                sem_s[b],
            )

        def wait_store(i, b):
            pltpu.make_async_copy(
                bufs[b],
                out_hbm.at[pl.ds(base + i * CHUNK, CHUNK)],
                sem_s[b],
            ).wait()

        def pick_targets(j, b):
            tv = tgt_v[pl.ds(j * CHUNK, CHUNK)]
            lanes = lax.broadcasted_iota(jnp.int32, (16,), 0)
            pacc = jnp.zeros((16,), jnp.float32)
            for r in range(CHUNK):
                t_r = tv[r]
                v = bufs[b][r, pl.ds((t_r // 16) * 16, 16)]
                u = lax.gather(
                    v,
                    jnp.full((16, 1), t_r % 16, jnp.int32),
                    lax.GatherDimensionNumbers(
                        offset_dims=(),
                        collapsed_slice_dims=(0,),
                        start_index_map=(0,),
                    ),
                    (1,),
                    mode=lax.GatherScatterMode.PROMISE_IN_BOUNDS,
                )
                pacc = jnp.where(lanes == r, u, pacc)
            picked_v[pl.ds(j * CHUNK, CHUNK)] = pacc

        gather_chunk(0, 0)

        N_MAIN = (N_CHUNKS - 1) // NBUF * NBUF

        def body(g, carry):
            for b in range(NBUF):
                j = g * NBUF + b
                bn = (b + 1) % NBUF
                wait_gather(j, b)
                store_chunk(j, b)
                pick_targets(j, b)

                @pl.when(j - (NBUF - 1) >= 0)
                def _():
                    wait_store(j - (NBUF - 1), bn)

                gather_chunk(j + 1, bn)

            return carry

        lax.fori_loop(0, N_MAIN // NBUF, body, 0)

        for j in range(N_MAIN, N_CHUNKS):
            b = j % NBUF
            wait_gather(j, b)
            store_chunk(j, b)
            pick_targets(j, b)
        pltpu.sync_copy(picked_v, picked_hbm.at[pl.ds(base, B_PER_W)])
        for j in range(N_CHUNKS - NBUF, N_CHUNKS):
            wait_store(j, j % NBUF)

    return gather_kernel(table, idx_flat, tgt_flat)


ROWS_BLK = 256
N_BLKS = N_ROWS // ROWS_BLK


def _tc_loss_kernel(picked_ref, logits_ref, acc_ref):
    i = pl.program_id(0)
    blk = logits_ref[...]
    m = jnp.max(blk, axis=1, keepdims=True)
    lse = jnp.log(jnp.sum(jnp.exp(blk - m), axis=1, keepdims=True)) + m
    part = jnp.sum(lse) - jnp.sum(picked_ref[0, 0, :])

    @pl.when(i == 0)
    def _():
        acc_ref[0, 0] = 0.0

    acc_ref[0, 0] += part

    @pl.when(i == N_BLKS - 1)
    def _():
        acc_ref[0, 0] = acc_ref[0, 0] / N_ROWS


def _tc_loss(logits, picked):
    tgt3 = picked.reshape(N_BLKS, 1, ROWS_BLK)
    acc = pl.pallas_call(
        _tc_loss_kernel,
        grid=(N_BLKS,),
        in_specs=[
            pl.BlockSpec((1, 1, ROWS_BLK), lambda i: (i, 0, 0)),
            pl.BlockSpec((ROWS_BLK, D), lambda i: (i, 0)),
        ],
        out_specs=pl.BlockSpec(
            (1, 1), lambda i: (0, 0), memory_space=pltpu.SMEM
        ),
        out_shape=jax.ShapeDtypeStruct((1, 1), jnp.float32),
    )(tgt3, logits)
    return acc[0, 0]


def kernel(x, targets, token_embedding_table):
    idx_flat = x.reshape(N_ROWS)
    tgt_flat = targets.reshape(N_ROWS)
    logits, picked = _sc_gather(token_embedding_table, idx_flat, tgt_flat)
    loss = _tc_loss(logits, picked)
    return (logits, loss)
